# baseline (speedup 1.0000x reference)
import jax
import jax.numpy as jnp
from jax import lax
from jax.experimental import pallas as pl
from jax.experimental.pallas import tpu as pltpu
from jax.experimental.pallas import tpu_sc as plsc

_LANES = 16


def _sc_time_emb_body(xm_ref, hour_ref, day_ref, month_ref, out_ref,
                      xm_v, rows_h, rows_d, rows_m, sem):
    cid = lax.axis_index("c")
    sid = lax.axis_index("s")

    @pl.when((cid == 0) & (sid == 0))
    def _():
        pltpu.sync_copy(xm_ref, xm_v)
        hour_idx = (xm_v[2] * 24.0).astype(jnp.int32)
        day_idx = (xm_v[1] * 32.0).astype(jnp.int32)
        month_idx = (xm_v[0] * 13.0).astype(jnp.int32)
        cp_h = pltpu.async_copy(hour_ref.at[hour_idx], rows_h, sem)
        cp_d = pltpu.async_copy(day_ref.at[day_idx], rows_d, sem)
        cp_m = pltpu.async_copy(month_ref.at[month_idx], rows_m, sem)
        cp_h.wait()
        cp_d.wait()
        cp_m.wait()
        b, d = rows_h.shape
        for r in range(b):
            for j in range(d // _LANES):
                s = pl.ds(j * _LANES, _LANES)
                rows_h[r, s] = rows_h[r, s] + rows_d[r, s] + rows_m[r, s]
        pltpu.sync_copy(rows_h, out_ref)


def _time_emb_sc(x_mark_t, hour_table, day_table, month_table):
    b = x_mark_t.shape[1]
    d = hour_table.shape[1]
    mesh = plsc.VectorSubcoreMesh(core_axis_name="c", subcore_axis_name="s")
    k = pl.kernel(
        _sc_time_emb_body,
        out_type=jax.ShapeDtypeStruct((b, d), jnp.float32),
        mesh=mesh,
        scratch_types=[
            pltpu.VMEM((3, b), jnp.float32),
            pltpu.VMEM((b, d), jnp.float32),
            pltpu.VMEM((b, d), jnp.float32),
            pltpu.VMEM((b, d), jnp.float32),
            pltpu.SemaphoreType.DMA,
        ],
    )
    return k(x_mark_t, hour_table, day_table, month_table)


_NBUF = 4
_CC = 2


def _stream_body(x_hbm, xm_ref, hour_ref, day_ref, month_ref,
                 np_ref, w1_ref, b1_ref, w2_ref, b2_ref,
                 out_hbm, node_scr, time_scr, in_bufs, out_bufs,
                 in_sems, out_sems):
    nch, cc, n, d = x_hbm.shape
    bsz = xm_ref.shape[0]
    per_b = nch // bsz

    for s in range(_NBUF):
        pltpu.make_async_copy(x_hbm.at[s], in_bufs.at[s], in_sems.at[s]
                              ).start()

    hour_idx = (xm_ref[:, 2:3] * 24.0).astype(jnp.int32)
    day_idx = (xm_ref[:, 1:2] * 32.0).astype(jnp.int32)
    month_idx = (xm_ref[:, 0:1] * 13.0).astype(jnp.int32)

    def _take(table_ref, idx):
        v = table_ref.shape[0]
        oh = (lax.broadcasted_iota(jnp.int32, (bsz, v), 1)
              == idx).astype(jnp.float32)
        return jnp.dot(oh, table_ref[...], preferred_element_type=jnp.float32)

    time_scr[...] = (_take(hour_ref, hour_idx) + _take(day_ref, day_idx)
                     + _take(month_ref, month_idx))
    time_ref = time_scr

    h = (np_ref[:, 0:1] * w1_ref[0:1, :]
         + np_ref[:, 1:2] * w1_ref[1:2, :]
         + np_ref[:, 2:3] * w1_ref[2:3, :]) + b1_ref[...]
    h = jnp.maximum(h, 0.0)
    node_scr[...] = (jnp.dot(h, w2_ref[...],
                             preferred_element_type=jnp.float32)
                     + b2_ref[...])

    def body(i, carry):
        slot = lax.rem(i, _NBUF)
        pltpu.make_async_copy(x_hbm.at[i], in_bufs.at[slot], in_sems.at[slot]
                              ).wait()
        b = lax.div(i, per_b)
        bias = node_scr[...] + time_ref[b]

        @pl.when(i >= _NBUF)
        def _():
            pltpu.make_async_copy(out_bufs.at[slot], out_hbm.at[i - _NBUF],
                                  out_sems.at[slot]).wait()

        out_bufs[slot] = in_bufs[slot] + bias[None]

        @pl.when(i + _NBUF < nch)
        def _():
            pltpu.make_async_copy(x_hbm.at[i + _NBUF], in_bufs.at[slot],
                                  in_sems.at[slot]).start()

        pltpu.make_async_copy(out_bufs.at[slot], out_hbm.at[i],
                              out_sems.at[slot]).start()
        return carry

    lax.fori_loop(0, nch, body, 0)

    for j in range(nch - _NBUF, nch):
        pltpu.make_async_copy(out_bufs.at[j % _NBUF], out_hbm.at[j],
                              out_sems.at[j % _NBUF]).wait()


def kernel(x, x_mark, node_pos, W1, b1, W2, b2,
           hour_table, day_table, month_table):
    bsz, c, n, d = x.shape

    nch = bsz * (c // _CC)
    x_r = x.reshape(nch, _CC, n, d)
    out = pl.pallas_call(
        _stream_body,
        in_specs=[
            pl.BlockSpec(memory_space=pltpu.MemorySpace.HBM),
            pl.BlockSpec(memory_space=pltpu.VMEM),
            pl.BlockSpec(memory_space=pltpu.VMEM),
            pl.BlockSpec(memory_space=pltpu.VMEM),
            pl.BlockSpec(memory_space=pltpu.VMEM),
            pl.BlockSpec(memory_space=pltpu.VMEM),
            pl.BlockSpec(memory_space=pltpu.VMEM),
            pl.BlockSpec(memory_space=pltpu.VMEM),
            pl.BlockSpec(memory_space=pltpu.VMEM),
            pl.BlockSpec(memory_space=pltpu.VMEM),
        ],
        out_specs=pl.BlockSpec(memory_space=pltpu.MemorySpace.HBM),
        out_shape=jax.ShapeDtypeStruct((nch, _CC, n, d), jnp.float32),
        scratch_shapes=[
            pltpu.VMEM((n, d), jnp.float32),
            pltpu.VMEM((bsz, d), jnp.float32),
            pltpu.VMEM((_NBUF, _CC, n, d), jnp.float32),
            pltpu.VMEM((_NBUF, _CC, n, d), jnp.float32),
            pltpu.SemaphoreType.DMA((_NBUF,)),
            pltpu.SemaphoreType.DMA((_NBUF,)),
        ],
    )(x_r, x_mark, hour_table, day_table, month_table, node_pos, W1,
      b1.reshape(1, d), W2, b2.reshape(1, d))
    return out.reshape(bsz, c, n, d)


# speedup vs baseline: 1.0498x; 1.0498x over previous
import jax
import jax.numpy as jnp
from jax import lax
from jax.experimental import pallas as pl
from jax.experimental.pallas import tpu as pltpu
from jax.experimental.pallas import tpu_sc as plsc

_LANES = 16


def _sc_time_emb_body(xm_ref, hour_ref, day_ref, month_ref, out_ref,
                      xm_v, rows_h, rows_d, rows_m, sem):
    cid = lax.axis_index("c")
    sid = lax.axis_index("s")

    @pl.when((cid == 0) & (sid == 0))
    def _():
        pltpu.sync_copy(xm_ref, xm_v)
        hour_idx = (xm_v[2] * 24.0).astype(jnp.int32)
        day_idx = (xm_v[1] * 32.0).astype(jnp.int32)
        month_idx = (xm_v[0] * 13.0).astype(jnp.int32)
        cp_h = pltpu.async_copy(hour_ref.at[hour_idx], rows_h, sem)
        cp_d = pltpu.async_copy(day_ref.at[day_idx], rows_d, sem)
        cp_m = pltpu.async_copy(month_ref.at[month_idx], rows_m, sem)
        cp_h.wait()
        cp_d.wait()
        cp_m.wait()
        b, d = rows_h.shape
        for r in range(b):
            for j in range(d // _LANES):
                s = pl.ds(j * _LANES, _LANES)
                rows_h[r, s] = rows_h[r, s] + rows_d[r, s] + rows_m[r, s]
        pltpu.sync_copy(rows_h, out_ref)


def _time_emb_sc(x_mark_t, hour_table, day_table, month_table):
    b = x_mark_t.shape[1]
    d = hour_table.shape[1]
    mesh = plsc.VectorSubcoreMesh(core_axis_name="c", subcore_axis_name="s")
    k = pl.kernel(
        _sc_time_emb_body,
        out_type=jax.ShapeDtypeStruct((b, d), jnp.float32),
        mesh=mesh,
        scratch_types=[
            pltpu.VMEM((3, b), jnp.float32),
            pltpu.VMEM((b, d), jnp.float32),
            pltpu.VMEM((b, d), jnp.float32),
            pltpu.VMEM((b, d), jnp.float32),
            pltpu.SemaphoreType.DMA,
        ],
    )
    return k(x_mark_t, hour_table, day_table, month_table)


_NBUF = 3
_CC = 8


def _stream_body(x_hbm, xm_ref, hour_ref, day_ref, month_ref,
                 np_ref, w1_ref, b1_ref, w2_ref, b2_ref,
                 out_hbm, node_scr, time_scr, in_bufs, out_bufs,
                 in_sems, out_sems):
    nch, cc, n, d = x_hbm.shape
    bsz = xm_ref.shape[0]
    per_b = nch // bsz

    for s in range(_NBUF):
        pltpu.make_async_copy(x_hbm.at[s], in_bufs.at[s], in_sems.at[s]
                              ).start()

    hour_idx = (xm_ref[:, 2:3] * 24.0).astype(jnp.int32)
    day_idx = (xm_ref[:, 1:2] * 32.0).astype(jnp.int32)
    month_idx = (xm_ref[:, 0:1] * 13.0).astype(jnp.int32)

    def _take(table_ref, idx):
        v = table_ref.shape[0]
        oh = (lax.broadcasted_iota(jnp.int32, (bsz, v), 1)
              == idx).astype(jnp.float32)
        return jnp.dot(oh, table_ref[...], preferred_element_type=jnp.float32)

    time_scr[...] = (_take(hour_ref, hour_idx) + _take(day_ref, day_idx)
                     + _take(month_ref, month_idx))
    time_ref = time_scr

    h = (np_ref[:, 0:1] * w1_ref[0:1, :]
         + np_ref[:, 1:2] * w1_ref[1:2, :]
         + np_ref[:, 2:3] * w1_ref[2:3, :]) + b1_ref[...]
    h = jnp.maximum(h, 0.0)
    node_scr[...] = (jnp.dot(h, w2_ref[...],
                             preferred_element_type=jnp.float32)
                     + b2_ref[...])

    def body(i, carry):
        slot = lax.rem(i, _NBUF)
        pltpu.make_async_copy(x_hbm.at[i], in_bufs.at[slot], in_sems.at[slot]
                              ).wait()
        b = lax.div(i, per_b)
        bias = node_scr[...] + time_ref[b]

        @pl.when(i >= _NBUF)
        def _():
            pltpu.make_async_copy(out_bufs.at[slot], out_hbm.at[i - _NBUF],
                                  out_sems.at[slot]).wait()

        out_bufs[slot] = in_bufs[slot] + bias[None]

        @pl.when(i + _NBUF < nch)
        def _():
            pltpu.make_async_copy(x_hbm.at[i + _NBUF], in_bufs.at[slot],
                                  in_sems.at[slot]).start()

        pltpu.make_async_copy(out_bufs.at[slot], out_hbm.at[i],
                              out_sems.at[slot]).start()
        return carry

    lax.fori_loop(0, nch, body, 0)

    for j in range(nch - _NBUF, nch):
        pltpu.make_async_copy(out_bufs.at[j % _NBUF], out_hbm.at[j],
                              out_sems.at[j % _NBUF]).wait()


def kernel(x, x_mark, node_pos, W1, b1, W2, b2,
           hour_table, day_table, month_table):
    bsz, c, n, d = x.shape

    nch = bsz * (c // _CC)
    x_r = x.reshape(nch, _CC, n, d)
    out = pl.pallas_call(
        _stream_body,
        in_specs=[
            pl.BlockSpec(memory_space=pltpu.MemorySpace.HBM),
            pl.BlockSpec(memory_space=pltpu.VMEM),
            pl.BlockSpec(memory_space=pltpu.VMEM),
            pl.BlockSpec(memory_space=pltpu.VMEM),
            pl.BlockSpec(memory_space=pltpu.VMEM),
            pl.BlockSpec(memory_space=pltpu.VMEM),
            pl.BlockSpec(memory_space=pltpu.VMEM),
            pl.BlockSpec(memory_space=pltpu.VMEM),
            pl.BlockSpec(memory_space=pltpu.VMEM),
            pl.BlockSpec(memory_space=pltpu.VMEM),
        ],
        out_specs=pl.BlockSpec(memory_space=pltpu.MemorySpace.HBM),
        out_shape=jax.ShapeDtypeStruct((nch, _CC, n, d), jnp.float32),
        scratch_shapes=[
            pltpu.VMEM((n, d), jnp.float32),
            pltpu.VMEM((bsz, d), jnp.float32),
            pltpu.VMEM((_NBUF, _CC, n, d), jnp.float32),
            pltpu.VMEM((_NBUF, _CC, n, d), jnp.float32),
            pltpu.SemaphoreType.DMA((_NBUF,)),
            pltpu.SemaphoreType.DMA((_NBUF,)),
        ],
    )(x_r, x_mark, hour_table, day_table, month_table, node_pos, W1,
      b1.reshape(1, d), W2, b2.reshape(1, d))
    return out.reshape(bsz, c, n, d)


# speedup vs baseline: 1.1278x; 1.0742x over previous
import jax
import jax.numpy as jnp
from jax import lax
from jax.experimental import pallas as pl
from jax.experimental.pallas import tpu as pltpu
from jax.experimental.pallas import tpu_sc as plsc

_LANES = 16


def _sc_time_emb_body(xm_ref, hour_ref, day_ref, month_ref, out_ref,
                      xm_v, rows_h, rows_d, rows_m, sem):
    cid = lax.axis_index("c")
    sid = lax.axis_index("s")

    @pl.when((cid == 0) & (sid == 0))
    def _():
        pltpu.sync_copy(xm_ref, xm_v)
        hour_idx = (xm_v[2] * 24.0).astype(jnp.int32)
        day_idx = (xm_v[1] * 32.0).astype(jnp.int32)
        month_idx = (xm_v[0] * 13.0).astype(jnp.int32)
        cp_h = pltpu.async_copy(hour_ref.at[hour_idx], rows_h, sem)
        cp_d = pltpu.async_copy(day_ref.at[day_idx], rows_d, sem)
        cp_m = pltpu.async_copy(month_ref.at[month_idx], rows_m, sem)
        cp_h.wait()
        cp_d.wait()
        cp_m.wait()
        b, d = rows_h.shape
        for r in range(b):
            for j in range(d // _LANES):
                s = pl.ds(j * _LANES, _LANES)
                rows_h[r, s] = rows_h[r, s] + rows_d[r, s] + rows_m[r, s]
        pltpu.sync_copy(rows_h, out_ref)


def _time_emb_sc(x_mark_t, hour_table, day_table, month_table):
    b = x_mark_t.shape[1]
    d = hour_table.shape[1]
    mesh = plsc.VectorSubcoreMesh(core_axis_name="c", subcore_axis_name="s")
    k = pl.kernel(
        _sc_time_emb_body,
        out_type=jax.ShapeDtypeStruct((b, d), jnp.float32),
        mesh=mesh,
        scratch_types=[
            pltpu.VMEM((3, b), jnp.float32),
            pltpu.VMEM((b, d), jnp.float32),
            pltpu.VMEM((b, d), jnp.float32),
            pltpu.VMEM((b, d), jnp.float32),
            pltpu.SemaphoreType.DMA,
        ],
    )
    return k(x_mark_t, hour_table, day_table, month_table)


_NBUF = 3
_CC = 8


def _stream_body(x_hbm, xmt_ref, hour_ref, day_ref, month_ref,
                 npt_ref, w1_ref, b1_ref, w2_ref, b2_ref,
                 out_hbm, node_scr, time_scr, in_bufs, out_bufs,
                 in_sems, out_sems):
    nch, cc, n, d = x_hbm.shape
    bsz = xmt_ref.shape[1]
    per_b = nch // bsz

    for s in range(_NBUF):
        pltpu.make_async_copy(x_hbm.at[s], in_bufs.at[s], in_sems.at[s]
                              ).start()

    hour_idx = (xmt_ref[2:3, :] * 24.0).astype(jnp.int32)
    day_idx = (xmt_ref[1:2, :] * 32.0).astype(jnp.int32)
    month_idx = (xmt_ref[0:1, :] * 13.0).astype(jnp.int32)

    def _take(table_ref, idx):
        v = table_ref.shape[0]
        oh_t = (lax.broadcasted_iota(jnp.int32, (v, bsz), 0)
                == idx).astype(jnp.float32)
        return lax.dot_general(oh_t, table_ref[...],
                               (((0,), (0,)), ((), ())),
                               preferred_element_type=jnp.float32)

    time_scr[...] = (_take(hour_ref, hour_idx) + _take(day_ref, day_idx)
                     + _take(month_ref, month_idx))
    time_ref = time_scr

    h = lax.dot_general(npt_ref[...], w1_ref[...],
                        (((0,), (0,)), ((), ())),
                        preferred_element_type=jnp.float32) + b1_ref[...]
    h = jnp.maximum(h, 0.0)
    node_scr[...] = (jnp.dot(h, w2_ref[...],
                             preferred_element_type=jnp.float32)
                     + b2_ref[...])

    def body(i, carry):
        slot = lax.rem(i, _NBUF)
        pltpu.make_async_copy(x_hbm.at[i], in_bufs.at[slot], in_sems.at[slot]
                              ).wait()
        b = lax.div(i, per_b)
        bias = node_scr[...] + time_ref[b]

        @pl.when(i >= _NBUF)
        def _():
            pltpu.make_async_copy(out_bufs.at[slot], out_hbm.at[i - _NBUF],
                                  out_sems.at[slot]).wait()

        out_bufs[slot] = in_bufs[slot] + bias[None]

        @pl.when(i + _NBUF < nch)
        def _():
            pltpu.make_async_copy(x_hbm.at[i + _NBUF], in_bufs.at[slot],
                                  in_sems.at[slot]).start()

        pltpu.make_async_copy(out_bufs.at[slot], out_hbm.at[i],
                              out_sems.at[slot]).start()
        return carry

    lax.fori_loop(0, nch, body, 0)

    for j in range(nch - _NBUF, nch):
        pltpu.make_async_copy(out_bufs.at[j % _NBUF], out_hbm.at[j],
                              out_sems.at[j % _NBUF]).wait()


def kernel(x, x_mark, node_pos, W1, b1, W2, b2,
           hour_table, day_table, month_table):
    bsz, c, n, d = x.shape

    nch = bsz * (c // _CC)
    x_r = x.reshape(nch, _CC, n, d)
    out = pl.pallas_call(
        _stream_body,
        in_specs=[
            pl.BlockSpec(memory_space=pltpu.MemorySpace.HBM),
            pl.BlockSpec(memory_space=pltpu.VMEM),
            pl.BlockSpec(memory_space=pltpu.VMEM),
            pl.BlockSpec(memory_space=pltpu.VMEM),
            pl.BlockSpec(memory_space=pltpu.VMEM),
            pl.BlockSpec(memory_space=pltpu.VMEM),
            pl.BlockSpec(memory_space=pltpu.VMEM),
            pl.BlockSpec(memory_space=pltpu.VMEM),
            pl.BlockSpec(memory_space=pltpu.VMEM),
            pl.BlockSpec(memory_space=pltpu.VMEM),
        ],
        out_specs=pl.BlockSpec(memory_space=pltpu.MemorySpace.HBM),
        out_shape=jax.ShapeDtypeStruct((nch, _CC, n, d), jnp.float32),
        scratch_shapes=[
            pltpu.VMEM((n, d), jnp.float32),
            pltpu.VMEM((bsz, d), jnp.float32),
            pltpu.VMEM((_NBUF, _CC, n, d), jnp.float32),
            pltpu.VMEM((_NBUF, _CC, n, d), jnp.float32),
            pltpu.SemaphoreType.DMA((_NBUF,)),
            pltpu.SemaphoreType.DMA((_NBUF,)),
        ],
    )(x_r, x_mark.T, hour_table, day_table, month_table, node_pos.T, W1,
      b1.reshape(1, d), W2, b2.reshape(1, d))
    return out.reshape(bsz, c, n, d)


# speedup vs baseline: 1.1339x; 1.0054x over previous
import jax
import jax.numpy as jnp
from jax import lax
from jax.experimental import pallas as pl
from jax.experimental.pallas import tpu as pltpu
from jax.experimental.pallas import tpu_sc as plsc

_LANES = 16


def _sc_time_emb_body(xm_ref, hour_ref, day_ref, month_ref, out_ref,
                      xm_v, rows_h, rows_d, rows_m, sem):
    cid = lax.axis_index("c")
    sid = lax.axis_index("s")

    @pl.when((cid == 0) & (sid == 0))
    def _():
        pltpu.sync_copy(xm_ref, xm_v)
        hour_idx = (xm_v[2] * 24.0).astype(jnp.int32)
        day_idx = (xm_v[1] * 32.0).astype(jnp.int32)
        month_idx = (xm_v[0] * 13.0).astype(jnp.int32)
        cp_h = pltpu.async_copy(hour_ref.at[hour_idx], rows_h, sem)
        cp_d = pltpu.async_copy(day_ref.at[day_idx], rows_d, sem)
        cp_m = pltpu.async_copy(month_ref.at[month_idx], rows_m, sem)
        cp_h.wait()
        cp_d.wait()
        cp_m.wait()
        b, d = rows_h.shape
        for r in range(b):
            for j in range(d // _LANES):
                s = pl.ds(j * _LANES, _LANES)
                rows_h[r, s] = rows_h[r, s] + rows_d[r, s] + rows_m[r, s]
        pltpu.sync_copy(rows_h, out_ref)


def _time_emb_sc(x_mark_t, hour_table, day_table, month_table):
    b = x_mark_t.shape[1]
    d = hour_table.shape[1]
    mesh = plsc.VectorSubcoreMesh(core_axis_name="c", subcore_axis_name="s")
    k = pl.kernel(
        _sc_time_emb_body,
        out_type=jax.ShapeDtypeStruct((b, d), jnp.float32),
        mesh=mesh,
        scratch_types=[
            pltpu.VMEM((3, b), jnp.float32),
            pltpu.VMEM((b, d), jnp.float32),
            pltpu.VMEM((b, d), jnp.float32),
            pltpu.VMEM((b, d), jnp.float32),
            pltpu.SemaphoreType.DMA,
        ],
    )
    return k(x_mark_t, hour_table, day_table, month_table)


_NBUF = 4
_CC = 8


def _stream_body(x_hbm, xmt_ref, hour_ref, day_ref, month_ref,
                 npt_ref, w1_ref, b1_ref, w2_ref, b2_ref,
                 out_hbm, node_scr, time_scr, in_bufs, out_bufs,
                 in_sems, out_sems):
    nch, cc, n, d = x_hbm.shape
    bsz = xmt_ref.shape[1]
    per_b = nch // bsz

    for s in range(_NBUF):
        pltpu.make_async_copy(x_hbm.at[s], in_bufs.at[s], in_sems.at[s]
                              ).start()

    hour_idx = (xmt_ref[2:3, :] * 24.0).astype(jnp.int32)
    day_idx = (xmt_ref[1:2, :] * 32.0).astype(jnp.int32)
    month_idx = (xmt_ref[0:1, :] * 13.0).astype(jnp.int32)

    def _take(table_ref, idx):
        v = table_ref.shape[0]
        oh_t = (lax.broadcasted_iota(jnp.int32, (v, bsz), 0)
                == idx).astype(jnp.float32)
        return lax.dot_general(oh_t, table_ref[...],
                               (((0,), (0,)), ((), ())),
                               preferred_element_type=jnp.float32)

    time_scr[...] = (_take(hour_ref, hour_idx) + _take(day_ref, day_idx)
                     + _take(month_ref, month_idx))
    time_ref = time_scr

    h = lax.dot_general(npt_ref[...], w1_ref[...],
                        (((0,), (0,)), ((), ())),
                        preferred_element_type=jnp.float32) + b1_ref[...]
    h = jnp.maximum(h, 0.0)
    node_scr[...] = (jnp.dot(h, w2_ref[...],
                             preferred_element_type=jnp.float32)
                     + b2_ref[...])

    def body(i, carry):
        slot = lax.rem(i, _NBUF)
        pltpu.make_async_copy(x_hbm.at[i], in_bufs.at[slot], in_sems.at[slot]
                              ).wait()
        b = lax.div(i, per_b)
        bias = node_scr[...] + time_ref[b]

        @pl.when(i >= _NBUF)
        def _():
            pltpu.make_async_copy(out_bufs.at[slot], out_hbm.at[i - _NBUF],
                                  out_sems.at[slot]).wait()

        out_bufs[slot] = in_bufs[slot] + bias[None]

        @pl.when(i + _NBUF < nch)
        def _():
            pltpu.make_async_copy(x_hbm.at[i + _NBUF], in_bufs.at[slot],
                                  in_sems.at[slot]).start()

        pltpu.make_async_copy(out_bufs.at[slot], out_hbm.at[i],
                              out_sems.at[slot]).start()
        return carry

    lax.fori_loop(0, nch, body, 0)

    for j in range(nch - _NBUF, nch):
        pltpu.make_async_copy(out_bufs.at[j % _NBUF], out_hbm.at[j],
                              out_sems.at[j % _NBUF]).wait()


def kernel(x, x_mark, node_pos, W1, b1, W2, b2,
           hour_table, day_table, month_table):
    bsz, c, n, d = x.shape

    nch = bsz * (c // _CC)
    x_r = x.reshape(nch, _CC, n, d)
    out = pl.pallas_call(
        _stream_body,
        in_specs=[
            pl.BlockSpec(memory_space=pltpu.MemorySpace.HBM),
            pl.BlockSpec(memory_space=pltpu.VMEM),
            pl.BlockSpec(memory_space=pltpu.VMEM),
            pl.BlockSpec(memory_space=pltpu.VMEM),
            pl.BlockSpec(memory_space=pltpu.VMEM),
            pl.BlockSpec(memory_space=pltpu.VMEM),
            pl.BlockSpec(memory_space=pltpu.VMEM),
            pl.BlockSpec(memory_space=pltpu.VMEM),
            pl.BlockSpec(memory_space=pltpu.VMEM),
            pl.BlockSpec(memory_space=pltpu.VMEM),
        ],
        out_specs=pl.BlockSpec(memory_space=pltpu.MemorySpace.HBM),
        out_shape=jax.ShapeDtypeStruct((nch, _CC, n, d), jnp.float32),
        scratch_shapes=[
            pltpu.VMEM((n, d), jnp.float32),
            pltpu.VMEM((bsz, d), jnp.float32),
            pltpu.VMEM((_NBUF, _CC, n, d), jnp.float32),
            pltpu.VMEM((_NBUF, _CC, n, d), jnp.float32),
            pltpu.SemaphoreType.DMA((_NBUF,)),
            pltpu.SemaphoreType.DMA((_NBUF,)),
        ],
    )(x_r, x_mark.T, hour_table, day_table, month_table, node_pos.T, W1,
      b1.reshape(1, d), W2, b2.reshape(1, d))
    return out.reshape(bsz, c, n, d)
